# baseline (speedup 1.0000x reference)
import functools

import jax
import jax.numpy as jnp
from jax import lax
from jax.experimental import pallas as pl
from jax.experimental.pallas import tpu as pltpu
from jax.experimental.pallas import tpu_sc as plsc

S, D, H, F, V = 2048, 1024, 16, 4096, 32000
DH = D // H

NC, NS = 2, 16
NW = NC * NS
ROWS_PER_W = S // NW

HP = H // 2
QC = 1024
NQ = S // QC
MC = 512
NM = S // MC
VT = 1280
NV = V // VT


def _ln(x, g, b):
    m = jnp.mean(x, axis=-1, keepdims=True)
    v = jnp.mean((x - m) ** 2, axis=-1, keepdims=True)
    return (x - m) * lax.rsqrt(v + 1e-5) * g + b


def _bf(x):
    return x.astype(jnp.bfloat16)


_sc_mesh = plsc.VectorSubcoreMesh(core_axis_name="c", subcore_axis_name="s",
                                  num_cores=NC, num_subcores=NS)


@functools.partial(
    pl.kernel,
    out_type=jax.ShapeDtypeStruct((S, D), jnp.float32),
    mesh=_sc_mesh,
    scratch_types=[
        pltpu.VMEM((ROWS_PER_W,), jnp.int32),
        pltpu.VMEM((ROWS_PER_W, D), jnp.float32),
        pltpu.SemaphoreType.DMA,
    ],
)
def _sc_gather(table_hbm, idx_hbm, out_hbm, idx_v, rows_v, sem):
    wid = lax.axis_index("s") * NC + lax.axis_index("c")
    base = wid * ROWS_PER_W
    pltpu.sync_copy(idx_hbm.at[pl.ds(base, ROWS_PER_W)], idx_v)
    pltpu.async_copy(table_hbm.at[idx_v], rows_v, sem).wait()
    pltpu.sync_copy(rows_v, out_hbm.at[pl.ds(base, ROWS_PER_W)])


def _attn_body(x_ref, g_ref, b_ref, wq_ref, wk_ref, wv_ref, o_ref,
               hln_s, k_s, v_s, bias_s):
    hp = pl.program_id(0)
    sq = pl.program_id(1)

    @pl.when(jnp.logical_and(hp == 0, sq == 0))
    def _():
        hln_s[...] = _bf(_ln(x_ref[...], g_ref[...], b_ref[...]))
        row = lax.broadcasted_iota(jnp.int32, (QC, QC), 0)
        col = lax.broadcasted_iota(jnp.int32, (QC, QC), 1)
        bias_s[...] = jnp.where(row >= col, jnp.float32(0), jnp.float32(-1e9))

    @pl.when(sq == 0)
    def _():
        hln = hln_s[...]
        k_s[...] = _bf(jnp.dot(hln, _bf(wk_ref[...]),
                               preferred_element_type=jnp.float32))
        v_s[...] = _bf(jnp.dot(hln, _bf(wv_ref[...]),
                               preferred_element_type=jnp.float32))

    qf = jnp.dot(hln_s[pl.ds(sq * QC, QC), :], _bf(wq_ref[...]),
                 preferred_element_type=jnp.float32)
    q2 = _bf(qf * (1.0 / (DH ** 0.5)))
    k2 = k_s[...]
    v2 = v_s[...]

    def head_ctx(i, with_prefix):
        q = q2[:, i * DH:(i + 1) * DH]
        kh = k2[:, i * DH:(i + 1) * DH]
        vh = v2[:, i * DH:(i + 1) * DH]
        kd = kh[QC:] if with_prefix else kh[:QC]
        sd = lax.dot_general(q, kd, (((1,), (1,)), ((), ())),
                             preferred_element_type=jnp.float32)
        sd = sd + bias_s[...]
        if not with_prefix:
            m = jnp.max(sd, axis=-1, keepdims=True)
            p = jnp.exp(sd - m)
            acc = jnp.dot(_bf(p), vh[:QC], preferred_element_type=jnp.float32)
            return acc / jnp.sum(p, axis=-1, keepdims=True)
        s0 = lax.dot_general(q, kh[:QC], (((1,), (1,)), ((), ())),
                             preferred_element_type=jnp.float32)
        m0 = jnp.max(s0, axis=-1, keepdims=True)
        m = jnp.maximum(m0, jnp.max(sd, axis=-1, keepdims=True))
        p0 = jnp.exp(s0 - m)
        pd = jnp.exp(sd - m)
        acc = (jnp.dot(_bf(p0), vh[:QC], preferred_element_type=jnp.float32)
               + jnp.dot(_bf(pd), vh[QC:], preferred_element_type=jnp.float32))
        l = (jnp.sum(p0, axis=-1, keepdims=True)
             + jnp.sum(pd, axis=-1, keepdims=True))
        return acc / l

    @pl.when(sq == 0)
    def _():
        o_ref[...] = _bf(jnp.concatenate(
            [head_ctx(0, False), head_ctx(1, False)], axis=1))

    @pl.when(sq == 1)
    def _():
        o_ref[...] = _bf(jnp.concatenate(
            [head_ctx(0, True), head_ctx(1, True)], axis=1))


_attn = pl.pallas_call(
    _attn_body,
    grid=(HP, NQ),
    in_specs=[
        pl.BlockSpec((S, D), lambda hp, sq: (0, 0)),
        pl.BlockSpec((1, D), lambda hp, sq: (0, 0)),
        pl.BlockSpec((1, D), lambda hp, sq: (0, 0)),
        pl.BlockSpec((D, 2 * DH), lambda hp, sq: (0, hp)),
        pl.BlockSpec((D, 2 * DH), lambda hp, sq: (0, hp)),
        pl.BlockSpec((D, 2 * DH), lambda hp, sq: (0, hp)),
    ],
    out_specs=pl.BlockSpec((QC, 2 * DH), lambda hp, sq: (sq, hp)),
    out_shape=jax.ShapeDtypeStruct((S, D), jnp.bfloat16),
    scratch_shapes=[
        pltpu.VMEM((S, D), jnp.bfloat16),
        pltpu.VMEM((S, 2 * DH), jnp.bfloat16),
        pltpu.VMEM((S, 2 * DH), jnp.bfloat16),
        pltpu.VMEM((QC, QC), jnp.float32),
    ],
)


def _pm_body(x_ref, ctx_ref, wo_ref, g2_ref, b2_ref, w1_ref, w2_ref,
             gf_ref, bf_ref, out_ref, x2_s):
    sc = pl.program_id(0)

    @pl.when(sc == 0)
    def _():
        x2_s[...] = x_ref[...] + jnp.dot(ctx_ref[...], _bf(wo_ref[...]),
                                         preferred_element_type=jnp.float32)

    x2 = x2_s[pl.ds(sc * MC, MC), :]
    h2 = _bf(_ln(x2, g2_ref[...], b2_ref[...]))
    t = jnp.dot(h2, w1_ref[...], preferred_element_type=jnp.float32)
    t = jax.nn.gelu(t)
    y = x2 + jnp.dot(_bf(t), w2_ref[...], preferred_element_type=jnp.float32)
    out_ref[...] = _bf(_ln(y, gf_ref[...], bf_ref[...]))


_projmlp = pl.pallas_call(
    _pm_body,
    grid=(NM,),
    in_specs=[
        pl.BlockSpec((S, D), lambda sc: (0, 0)),
        pl.BlockSpec((S, D), lambda sc: (0, 0)),
        pl.BlockSpec((D, D), lambda sc: (0, 0)),
        pl.BlockSpec((1, D), lambda sc: (0, 0)),
        pl.BlockSpec((1, D), lambda sc: (0, 0)),
        pl.BlockSpec((D, F), lambda sc: (0, 0)),
        pl.BlockSpec((F, D), lambda sc: (0, 0)),
        pl.BlockSpec((1, D), lambda sc: (0, 0)),
        pl.BlockSpec((1, D), lambda sc: (0, 0)),
    ],
    out_specs=pl.BlockSpec((MC, D), lambda sc: (sc, 0)),
    out_shape=jax.ShapeDtypeStruct((S, D), jnp.bfloat16),
    scratch_shapes=[pltpu.VMEM((S, D), jnp.float32)],
)


def _logits_body(hf_ref, we_ref, out_ref):
    out_ref[...] = lax.dot_general(
        hf_ref[...], _bf(we_ref[...]), (((1,), (1,)), ((), ())),
        preferred_element_type=jnp.float32)


_logits = pl.pallas_call(
    _logits_body,
    grid=(NV,),
    in_specs=[
        pl.BlockSpec((S, D), lambda vt: (0, 0)),
        pl.BlockSpec((VT, D), lambda vt: (vt, 0)),
    ],
    out_specs=pl.BlockSpec((S, VT), lambda vt: (0, vt)),
    out_shape=jax.ShapeDtypeStruct((S, V), jnp.float32),
)


def kernel(input_ids, W_emb, Wq, Wk, Wv, Wo, W1, W2,
           ln1_g, ln1_b, ln2_g, ln2_b, lnf_g, lnf_b):
    ids = input_ids.reshape(S).astype(jnp.int32)
    x = _sc_gather(W_emb, ids)
    ctx = _attn(x, ln1_g.reshape(1, D), ln1_b.reshape(1, D),
                Wq, Wk, Wv)
    hf = _projmlp(x, ctx, Wo, ln2_g.reshape(1, D), ln2_b.reshape(1, D),
                  _bf(W1), _bf(W2), lnf_g.reshape(1, D), lnf_b.reshape(1, D))
    logits = _logits(hf, W_emb)
    return logits.reshape(1, S, V)
